# Initial kernel scaffold; baseline (speedup 1.0000x reference)
import jax
import jax.numpy as jnp
from jax.experimental import pallas as pl


def kernel(edge_index, review_feat, user_cj, user_ci, movie_cj, movie_ci, W_user, W_movie, prob_w_um, score_w_um, review_w_um, prob_w_mu, score_w_mu, review_w_mu, ufc_W, ufc_b, ifc_W, ifc_b):
    raise NotImplementedError("write your pallas kernel here")


# speedup vs baseline: 1.0158x; 1.0158x over previous
import jax
import jax.numpy as jnp
from jax.experimental import pallas as pl
from jax.experimental.pallas import tpu as pltpu

NU = 10000
NM = 10000
D = 128
R = 5
E = 100000


def _gelu_exact(x):
    return x * 0.5 * (1.0 + jax.lax.erf(x * 0.7071067811865476))


def _tail_body(uf_ref, if_ref, uci_ref, ici_ref, uW_ref, ub_ref, iW_ref, ib_ref,
               uo_ref, io_ref):
    uf = _gelu_exact(uf_ref[...] * uci_ref[...])
    io = _gelu_exact(if_ref[...] * ici_ref[...])
    uo_ref[...] = uf @ uW_ref[...].T + ub_ref[...][None, :]
    io_ref[...] = io @ iW_ref[...].T + ib_ref[...][None, :]


def kernel(edge_index, review_feat, user_cj, user_ci, movie_cj, movie_ci,
           W_user, W_movie, prob_w_um, score_w_um, review_w_um,
           prob_w_mu, score_w_mu, review_w_mu, ufc_W, ufc_b, ifc_W, ifc_b):
    ufeat = jnp.zeros((NU, D), dtype=jnp.float32)
    ifeat = jnp.zeros((NM, D), dtype=jnp.float32)
    for r in range(R):
        src = edge_index[r, 0]
        dst = edge_index[r, 1]
        rfeat = review_feat[r]
        pa = jax.nn.sigmoid(rfeat @ prob_w_um[r])[:, None]
        rf = (rfeat @ review_w_um[r].T) * jax.nn.sigmoid(rfeat @ score_w_um[r])[:, None]
        m = (jnp.take(W_user[r], src, axis=0) * pa + rf) * jnp.take(user_cj, src, axis=0)
        ifeat = ifeat + jax.ops.segment_sum(m, dst, num_segments=NM)
        pa2 = jax.nn.sigmoid(rfeat @ prob_w_mu[r])[:, None]
        rf2 = (rfeat @ review_w_mu[r].T) * jax.nn.sigmoid(rfeat @ score_w_mu[r])[:, None]
        m2 = (jnp.take(W_movie[r], dst, axis=0) * pa2 + rf2) * jnp.take(movie_cj, dst, axis=0)
        ufeat = ufeat + jax.ops.segment_sum(m2, src, num_segments=NU)
    grid = 10
    blk_u = NU // grid
    blk_m = NM // grid
    uo, io = pl.pallas_call(
        _tail_body,
        grid=(grid,),
        in_specs=[
            pl.BlockSpec((blk_u, D), lambda i: (i, 0)),
            pl.BlockSpec((blk_m, D), lambda i: (i, 0)),
            pl.BlockSpec((blk_u, 1), lambda i: (i, 0)),
            pl.BlockSpec((blk_m, 1), lambda i: (i, 0)),
            pl.BlockSpec((D, D), lambda i: (0, 0)),
            pl.BlockSpec((D,), lambda i: (0,)),
            pl.BlockSpec((D, D), lambda i: (0, 0)),
            pl.BlockSpec((D,), lambda i: (0,)),
        ],
        out_specs=[
            pl.BlockSpec((blk_u, D), lambda i: (i, 0)),
            pl.BlockSpec((blk_m, D), lambda i: (i, 0)),
        ],
        out_shape=[
            jax.ShapeDtypeStruct((NU, D), jnp.float32),
            jax.ShapeDtypeStruct((NM, D), jnp.float32),
        ],
    )(ufeat, ifeat, user_ci, movie_ci, ufc_W, ufc_b, ifc_W, ifc_b)
    return (uo, io)


# speedup vs baseline: 3.8367x; 3.7770x over previous
import functools

import jax
import jax.numpy as jnp
from jax import lax
from jax.experimental import pallas as pl
from jax.experimental.pallas import tpu as pltpu
from jax.experimental.pallas import tpu_sc as plsc

NU = 10000
NM = 10000
D = 128
R = 5
E = 100000
N = R * E
K = 80
CH = N // K
NS = 16
TRIPS = (CH + NS - 1) // NS
ROWS_PER_TILE = 624


def _dense_body(rfeat_ref, pwu_ref, swu_ref, rwu_ref, pwm_ref, swm_ref, rwm_ref,
                rf0_ref, pa0_ref, rf1_ref, pa1_ref):
    x = rfeat_ref[0]
    for rw_ref, sw_ref, pw_ref, rf_ref, pa_ref in (
            (rwu_ref, swu_ref, pwu_ref, rf0_ref, pa0_ref),
            (rwm_ref, swm_ref, pwm_ref, rf1_ref, pa1_ref)):
        rw = rw_ref[0]
        rf = lax.dot_general(x, rw, (((1,), (1,)), ((), ())),
                             preferred_element_type=jnp.float32)
        sg = jax.nn.sigmoid(x @ sw_ref[0, 0])
        pa = jax.nn.sigmoid(x @ pw_ref[0, 0])
        rf_ref[0] = rf * sg[:, None]
        pa_ref[0] = jnp.broadcast_to(pa[:, None], pa.shape + (16,))


def _dense_phase(review_feat, prob_w_um, score_w_um, review_w_um,
                 prob_w_mu, score_w_mu, review_w_mu):
    be = 1000
    grid = (R, E // be)
    return pl.pallas_call(
        _dense_body,
        grid=grid,
        in_specs=[
            pl.BlockSpec((1, be, D), lambda r, b: (r, b, 0)),
            pl.BlockSpec((1, 1, D), lambda r, b: (r, 0, 0)),
            pl.BlockSpec((1, 1, D), lambda r, b: (r, 0, 0)),
            pl.BlockSpec((1, D, D), lambda r, b: (r, 0, 0)),
            pl.BlockSpec((1, 1, D), lambda r, b: (r, 0, 0)),
            pl.BlockSpec((1, 1, D), lambda r, b: (r, 0, 0)),
            pl.BlockSpec((1, D, D), lambda r, b: (r, 0, 0)),
        ],
        out_specs=[
            pl.BlockSpec((1, be, D), lambda r, b: (r, b, 0)),
            pl.BlockSpec((1, be, 16), lambda r, b: (r, b, 0)),
            pl.BlockSpec((1, be, D), lambda r, b: (r, b, 0)),
            pl.BlockSpec((1, be, 16), lambda r, b: (r, b, 0)),
        ],
        out_shape=[
            jax.ShapeDtypeStruct((R, E, D), jnp.float32),
            jax.ShapeDtypeStruct((R, E, 16), jnp.float32),
            jax.ShapeDtypeStruct((R, E, D), jnp.float32),
            jax.ShapeDtypeStruct((R, E, 16), jnp.float32),
        ],
    )(review_feat, prob_w_um[:, None, :], score_w_um[:, None, :], review_w_um,
      prob_w_mu[:, None, :], score_w_mu[:, None, :], review_w_mu)


def _sc_body(Wu, Wm, ucj, mcj,
             widx0, cidx0, sidx0, pa0, rf0,
             widx1, cidx1, sidx1, pa1, rf1,
             ufeat_out, ifeat_out,
             widx_v, cidx_v, sidx_v, pa_v, cj_v, rf_v, w_v, zb_v,
             acc, sem_g, sem_c):
    c = lax.axis_index("c")
    s = lax.axis_index("s")

    def _zrow(i, carry):
        for l in range(8):
            zb_v[i, pl.ds(l * 16, 16)] = jnp.zeros((16,), jnp.float32)
        return carry
    lax.fori_loop(0, 16, _zrow, 0)
    ntrips = jnp.where(s == NS - 1, 40, 39)

    def _zcopy(j, carry):
        pltpu.sync_copy(zb_v, acc.at[pl.ds(s * ROWS_PER_TILE + j * 16, 16)])
        return carry
    lax.fori_loop(0, ntrips, _zcopy, 0)
    plsc.subcore_barrier()

    def _process(widx_hbm, cidx_hbm, sidx_hbm, pa_hbm, rf_hbm,
                 wtab_hbm, cj_hbm):
        def _chunk_trip(n, carry):
            ci = s + n * NS

            @pl.when(ci < CH)
            def _():
                base = ci * K
                pltpu.sync_copy(widx_hbm.at[pl.ds(base, K)], widx_v)
                pltpu.sync_copy(cidx_hbm.at[pl.ds(base, K)], cidx_v)
                pltpu.sync_copy(sidx_hbm.at[pl.ds(base, K)], sidx_v)
                pltpu.sync_copy(pa_hbm.at[pl.ds(base, K)], pa_v)
                pltpu.sync_copy(rf_hbm.at[pl.ds(base, K)], rf_v)

                g1 = pltpu.async_copy(wtab_hbm.at[widx_v], w_v, sem_g)
                g2 = pltpu.async_copy(cj_hbm.at[cidx_v], cj_v, sem_c)
                g1.wait()
                g2.wait()

                def _edge(e, carry2):
                    pa_s = pa_v[e, :]
                    cj_s = cj_v[e, pl.ds(0, 16)]
                    for l in range(8):
                        sl = pl.ds(l * 16, 16)
                        w_v[e, sl] = (w_v[e, sl] * pa_s + rf_v[e, sl]) * cj_s
                    return carry2
                lax.fori_loop(0, K, _edge, 0)

                pltpu.sync_copy(w_v, acc.at[sidx_v], add=True)
            return carry
        lax.fori_loop(0, TRIPS, _chunk_trip, 0)

    @pl.when(c == 0)
    def _():
        _process(widx0, cidx0, sidx0, pa0, rf0, Wu, ucj)

    @pl.when(c == 1)
    def _():
        _process(widx1, cidx1, sidx1, pa1, rf1, Wm, mcj)

    plsc.subcore_barrier()

    @pl.when(c == 0)
    def _():
        def _fcopy(j, carry):
            off = s * ROWS_PER_TILE + j * 16
            pltpu.sync_copy(acc.at[pl.ds(off, 16)],
                            ifeat_out.at[pl.ds(off, 16)])
            return carry
        lax.fori_loop(0, ntrips, _fcopy, 0)

    @pl.when(c == 1)
    def _():
        def _fcopy(j, carry):
            off = s * ROWS_PER_TILE + j * 16
            pltpu.sync_copy(acc.at[pl.ds(off, 16)],
                            ufeat_out.at[pl.ds(off, 16)])
            return carry
        lax.fori_loop(0, ntrips, _fcopy, 0)


def _sparse_phase(Wu, Wm, ucj, mcj, d0, d1):
    mesh = plsc.VectorSubcoreMesh(core_axis_name="c", subcore_axis_name="s")
    fn = pl.kernel(
        _sc_body,
        out_type=(jax.ShapeDtypeStruct((NU, D), jnp.float32),
                  jax.ShapeDtypeStruct((NM, D), jnp.float32)),
        mesh=mesh,
        scratch_types=[
            pltpu.VMEM((K,), jnp.int32),
            pltpu.VMEM((K,), jnp.int32),
            pltpu.VMEM((K,), jnp.int32),
            pltpu.VMEM((K, 16), jnp.float32),
            pltpu.VMEM((K, D), jnp.float32),
            pltpu.VMEM((K, D), jnp.float32),
            pltpu.VMEM((K, D), jnp.float32),
            pltpu.VMEM((16, D), jnp.float32),
            pltpu.VMEM_SHARED((NU, D), jnp.float32),
            pltpu.SemaphoreType.DMA,
            pltpu.SemaphoreType.DMA,
        ],
    )
    return fn(Wu, Wm, ucj, mcj, *d0, *d1)


def _gelu_exact(x):
    return x * 0.5 * (1.0 + lax.erf(x * 0.7071067811865476))


def _tail_body(uf_ref, if_ref, uci_ref, ici_ref, uW_ref, ub_ref, iW_ref, ib_ref,
               uo_ref, io_ref):
    uf = _gelu_exact(uf_ref[...] * uci_ref[...])
    io = _gelu_exact(if_ref[...] * ici_ref[...])
    uo_ref[...] = uf @ uW_ref[...].T + ub_ref[...][None, :]
    io_ref[...] = io @ iW_ref[...].T + ib_ref[...][None, :]


def _tail_phase(ufeat, ifeat, user_ci, movie_ci, ufc_W, ufc_b, ifc_W, ifc_b):
    grid = 10
    blk_u = NU // grid
    blk_m = NM // grid
    return pl.pallas_call(
        _tail_body,
        grid=(grid,),
        in_specs=[
            pl.BlockSpec((blk_u, D), lambda i: (i, 0)),
            pl.BlockSpec((blk_m, D), lambda i: (i, 0)),
            pl.BlockSpec((blk_u, 1), lambda i: (i, 0)),
            pl.BlockSpec((blk_m, 1), lambda i: (i, 0)),
            pl.BlockSpec((D, D), lambda i: (0, 0)),
            pl.BlockSpec((D,), lambda i: (0,)),
            pl.BlockSpec((D, D), lambda i: (0, 0)),
            pl.BlockSpec((D,), lambda i: (0,)),
        ],
        out_specs=[
            pl.BlockSpec((blk_u, D), lambda i: (i, 0)),
            pl.BlockSpec((blk_m, D), lambda i: (i, 0)),
        ],
        out_shape=[
            jax.ShapeDtypeStruct((NU, D), jnp.float32),
            jax.ShapeDtypeStruct((NM, D), jnp.float32),
        ],
    )(ufeat, ifeat, user_ci, movie_ci, ufc_W, ufc_b, ifc_W, ifc_b)


def kernel(edge_index, review_feat, user_cj, user_ci, movie_cj, movie_ci,
           W_user, W_movie, prob_w_um, score_w_um, review_w_um,
           prob_w_mu, score_w_mu, review_w_mu, ufc_W, ufc_b, ifc_W, ifc_b):
    rf0, pa0, rf1, pa1 = _dense_phase(
        review_feat, prob_w_um, score_w_um, review_w_um,
        prob_w_mu, score_w_mu, review_w_mu)

    src = edge_index[:, 0, :].astype(jnp.int32)
    dst = edge_index[:, 1, :].astype(jnp.int32)
    roffs = (jnp.arange(R, dtype=jnp.int32) * NU)[:, None]
    d0 = ((src + roffs).reshape(N), src.reshape(N), dst.reshape(N),
          pa0.reshape(N, 16), rf0.reshape(N, D))
    d1 = ((dst + roffs).reshape(N), dst.reshape(N), src.reshape(N),
          pa1.reshape(N, 16), rf1.reshape(N, D))

    ufeat, ifeat = _sparse_phase(
        W_user.reshape(R * NU, D), W_movie.reshape(R * NM, D),
        jnp.broadcast_to(user_cj.reshape(NU, 1), (NU, D)),
        jnp.broadcast_to(movie_cj.reshape(NM, 1), (NM, D)), d0, d1)

    return _tail_phase(ufeat, ifeat, user_ci, movie_ci,
                       ufc_W, ufc_b, ifc_W, ifc_b)
